# baseline (speedup 1.0000x reference)
import functools

import jax
import jax.numpy as jnp
from jax import lax
from jax.experimental import pallas as pl
from jax.experimental.pallas import tpu as pltpu
from jax.experimental.pallas import tpu_sc as plsc

NC = 2
NS = 16
NW = NC * NS
LANES = 16


@functools.partial(jax.jit, static_argnums=(3, 4))
def _sc_embed(idx_flat, token_table, pos_table, B, L):
    V, D = token_table.shape
    N = B * L
    rows_per_w = N // NW
    group = 2 * L
    ngroups = rows_per_w // group
    chunk = 80
    nchunks = group // chunk

    mesh = plsc.VectorSubcoreMesh(core_axis_name="c", subcore_axis_name="s")

    @functools.partial(
        pl.kernel,
        out_type=jax.ShapeDtypeStruct((N, D), jnp.float32),
        mesh=mesh,
        compiler_params=pltpu.CompilerParams(use_tc_tiling_on_sc=False),
        scratch_types=[
            pltpu.VMEM((group,), jnp.int32),
            pltpu.VMEM((group, D), jnp.float32),
            pltpu.VMEM((L, D), jnp.float32),
            pltpu.SemaphoreType.DMA,
        ],
    )
    def body(idx_hbm, tok_hbm, pos_hbm, out_hbm, idx_v, rows_v, pos_v, sem_g):
        c = lax.axis_index("c")
        s = lax.axis_index("s")
        wid = s * NC + c
        base = wid * rows_per_w
        pltpu.sync_copy(pos_hbm, pos_v)

        def do_group(g, carry):
            off = pl.multiple_of(base + g * group, group)
            pltpu.sync_copy(idx_hbm.at[pl.ds(off, group)], idx_v)
            cps = []
            for cc in range(nchunks):
                cps.append(pltpu.async_copy(
                    tok_hbm.at[idx_v.at[pl.ds(cc * chunk, chunk)]],
                    rows_v.at[pl.ds(cc * chunk, chunk)],
                    sem_g))
            for cp in cps:
                cp.wait()

            def add_row(r, carry2):
                for q in range(D // LANES):
                    sl = pl.ds(q * LANES, LANES)
                    pv = pos_v[r, sl]
                    rows_v[r, sl] = rows_v[r, sl] + pv
                    rows_v[r + L, sl] = rows_v[r + L, sl] + pv
                return carry2

            lax.fori_loop(0, L, add_row, 0)
            pltpu.sync_copy(rows_v, out_hbm.at[pl.ds(off, group)])
            return carry

        lax.fori_loop(0, ngroups, do_group, 0)

    return body(idx_flat, token_table, pos_table)


def kernel(inputs, token_table, pos_table):
    B, L = inputs.shape
    D = token_table.shape[1]
    idx_flat = inputs.reshape(B * L).astype(jnp.int32)
    out = _sc_embed(idx_flat, token_table, pos_table, B, L)
    return out.reshape(B, L, D)
